# Initial kernel scaffold; baseline (speedup 1.0000x reference)
import jax
import jax.numpy as jnp
from jax.experimental import pallas as pl


def kernel(x, edge_index, gamma, beta, W1, b1, W2, b2):
    raise NotImplementedError("write your pallas kernel here")


# speedup vs baseline: 2.4353x; 2.4353x over previous
import jax
import jax.numpy as jnp
from jax.experimental import pallas as pl
from jax.experimental.pallas import tpu as pltpu

_N = 10000
_D = 256
_EPS = 1e-5


def _bn_mm(x, g, b, W1):
    def body(x_ref, g_ref, b_ref, w_ref, o_ref):
        xv = x_ref[...]
        mean = jnp.mean(xv, 0, keepdims=True)
        xm = xv - mean
        var = jnp.mean(xm * xm, 0, keepdims=True)
        h = xm * (g_ref[...] * jax.lax.rsqrt(var + _EPS)) + b_ref[...]
        o_ref[...] = jnp.dot(h, w_ref[...], preferred_element_type=jnp.float32)

    return pl.pallas_call(
        body,
        out_shape=jax.ShapeDtypeStruct((_N, _D), jnp.float32),
    )(x, g, b, W1)


def _final(acc, zt, dis2, ss2, b1, W2, b2):
    def body(a_ref, z_ref, d_ref, s_ref, b1_ref, w2_ref, b2_ref, o_ref):
        d = d_ref[...]
        out1 = d * (a_ref[...] + z_ref[...]) + b1_ref[...]
        h1 = jnp.where(out1 > 0, out1, 0.1 * out1)
        w = d * (d + s_ref[...])
        pooled = jnp.sum(h1 * w, axis=0, keepdims=True) * (1.0 / _N)
        o_ref[...] = (
            jnp.dot(pooled, w2_ref[...], preferred_element_type=jnp.float32)
            + b2_ref[...]
        )

    return pl.pallas_call(
        body,
        out_shape=jax.ShapeDtypeStruct((1, _D), jnp.float32),
    )(acc, zt, dis2, ss2, b1, W2, b2)


def kernel(x, edge_index, gamma, beta, W1, b1, W2, b2):
    src = edge_index[0].astype(jnp.int32)
    dst = edge_index[1].astype(jnp.int32)
    g = gamma.reshape(1, _D)
    b = beta.reshape(1, _D)
    b1r = b1.reshape(1, _D)
    b2r = b2.reshape(1, _D)

    y = _bn_mm(x, g, b, W1)

    cnt = jnp.zeros((_N,), jnp.float32).at[dst].add(1.0)
    dis = jax.lax.rsqrt(cnt + 1.0)
    zt = dis[:, None] * y
    acc = jnp.zeros((_N, _D), jnp.float32).at[dst].add(zt[src])
    ss = jnp.zeros((_N,), jnp.float32).at[src].add(dis[dst])

    return _final(acc, zt, dis.reshape(_N, 1), ss.reshape(_N, 1), b1r, W2, b2r)


# speedup vs baseline: 2.4364x; 1.0005x over previous
import jax
import jax.numpy as jnp
from jax.experimental import pallas as pl

_N = 10000
_D = 256
_E = 160000
_EPS = 1e-5

def _bn_mm(x, g, b, W1):
    def body(x_ref, g_ref, b_ref, w_ref, o_ref):
        xv = x_ref[...]
        mean = jnp.mean(xv, 0, keepdims=True)
        xm = xv - mean
        var = jnp.mean(xm * xm, 0, keepdims=True)
        h = xm * (g_ref[...] * jax.lax.rsqrt(var + _EPS)) + b_ref[...]
        o_ref[...] = jnp.dot(h, w_ref[...], preferred_element_type=jnp.float32)

    return pl.pallas_call(
        body,
        out_shape=jax.ShapeDtypeStruct((_N, _D), jnp.float32),
    )(x, g, b, W1)


def _final(acc, zt, dis2, ss2, b1, W2, b2):
    def body(a_ref, z_ref, d_ref, s_ref, b1_ref, w2_ref, b2_ref, o_ref):
        d = d_ref[...]
        out1 = d * (a_ref[...] + z_ref[...]) + b1_ref[...]
        h1 = jnp.where(out1 > 0, out1, 0.1 * out1)
        w = d * (d + s_ref[...])
        pooled = jnp.sum(h1 * w, axis=0, keepdims=True) * (1.0 / _N)
        o_ref[...] = (
            jnp.dot(pooled, w2_ref[...], preferred_element_type=jnp.float32)
            + b2_ref[...]
        )

    return pl.pallas_call(
        body,
        out_shape=jax.ShapeDtypeStruct((1, _D), jnp.float32),
    )(acc, zt, dis2, ss2, b1, W2, b2)


def kernel(x, edge_index, gamma, beta, W1, b1, W2, b2):
    src = edge_index[0].astype(jnp.int32)
    dst = edge_index[1].astype(jnp.int32)
    g = gamma.reshape(1, _D)
    b = beta.reshape(1, _D)
    b1r = b1.reshape(1, _D)
    b2r = b2.reshape(1, _D)

    y = _bn_mm(x, g, b, W1)

    cnt = jnp.zeros((_N,), jnp.float32).at[dst].add(1.0)
    dis = jax.lax.rsqrt(cnt + 1.0)
    zt = dis[:, None] * y
    acc = jnp.zeros((_N, _D), jnp.float32).at[dst].add(zt[src])
    ss = jnp.zeros((_N,), jnp.float32).at[src].add(dis[dst])

    return _final(acc, zt, dis.reshape(_N, 1), ss.reshape(_N, 1), b1r, W2, b2r)
